# baseline (speedup 1.0000x reference)
import functools

import jax
import jax.numpy as jnp
from jax import lax
from jax.experimental import pallas as pl
from jax.experimental.pallas import tpu as pltpu
from jax.experimental.pallas import tpu_sc as plsc

_NC = 2
_NS = 16
_L = 16
_NW = _NC * _NS
_CH = 128
_NBUF = 2


def _partials_body(num_classes, rpw, yp_hbm, yt_hbm, tci_hbm, out_hbm,
                   tci_v, idx_v, yt_v, bufs, acc_v, sem0, sem1):
    c = lax.axis_index("c")
    s = lax.axis_index("s")
    wid = s * _NC + c
    base = wid * rpw

    pltpu.sync_copy(tci_hbm.at[pl.ds(base, rpw)], tci_v)
    pltpu.sync_copy(yt_hbm.at[pl.ds(base * 4, rpw * 4)], yt_v)

    lanes = lax.iota(jnp.int32, _L)
    nch = rpw // _CH
    sems = [sem0, sem1]

    def build(i, carry):
        t = tci_v[pl.ds(i * _L, _L)]
        rows = (base + i * _L) + lanes
        f4 = (rows * num_classes + t) * 4
        j = i // (_CH // _L)
        k = i % (_CH // _L)
        idx_v[j, pl.ds(k * _L, _L)] = lax.shift_right_logical(f4, 7)
        return carry

    lax.fori_loop(0, rpw // _L, build, 0)

    def fire(j):
        pltpu.async_copy(yp_hbm.at[idx_v.at[j]], bufs.at[j % _NBUF],
                         sems[j % _NBUF])

    def drain(j):
        pltpu.make_async_copy(yp_hbm.at[idx_v.at[j]], bufs.at[j % _NBUF],
                              sems[j % _NBUF]).wait()

    row_off = lax.shift_right_logical(lanes, 2)
    col = lanes & 3

    fire(0)
    lacc = jnp.zeros((_L,), jnp.float32)
    cacc = jnp.zeros((_L,), jnp.float32)
    for j in range(nch):
        if j + 1 < nch:
            fire(j + 1)
        drain(j)
        buf = bufs.at[j % _NBUF]

        def step(i, carry, j=j, buf=buf):
            la, ca = carry
            rows16 = i * 4 + row_off
            grow16 = j * _CH + rows16
            t16 = plsc.load_gather(tci_v, [grow16])
            f4 = ((base + grow16) * num_classes + t16) * 4
            o16 = (f4 & 127) + col
            pb16 = plsc.load_gather(buf, [rows16, o16])
            tb16 = yt_v[pl.ds(j * _CH * 4 + i * _L, _L)]
            diff = jnp.abs(tb16 - pb16)
            loss = jnp.where(diff < 1.0, 0.5 * diff * diff, diff - 0.5)
            m = t16 > 0
            la = la + jnp.where(m, loss, 0.0)
            ca = ca + jnp.where(m, 1.0, 0.0)
            return la, ca

        lacc, cacc = lax.fori_loop(0, _CH // 4, step, (lacc, cacc))

    acc_v[0, :] = lacc
    acc_v[1, :] = cacc
    pltpu.sync_copy(acc_v, out_hbm.at[wid])


def _reduce_body(parts_hbm, out_hbm, parts_v, res_v):
    c = lax.axis_index("c")
    s = lax.axis_index("s")

    @pl.when(jnp.logical_and(s == 0, c == 0))
    def _():
        pltpu.sync_copy(parts_hbm, parts_v)
        zero = jnp.zeros((_L,), jnp.float32)

        def red(i, carry):
            ls, cs = carry
            return ls + parts_v[i, 0, :], cs + parts_v[i, 1, :]

        ls, cs = lax.fori_loop(0, _NW, red, (zero, zero))
        tv = zero + jnp.sum(ls)
        cv = zero + jnp.sum(cs)
        mean = tv / jnp.maximum(cv, 1.0)
        res_v[:] = jnp.where(cv > 0.0, mean, zero)
        pltpu.sync_copy(res_v, out_hbm)


@functools.partial(jax.jit, static_argnums=(3, 4))
def _sc_loss(yp, yt, tci, num_rows, num_classes):
    rpw = num_rows // _NW
    mesh = plsc.VectorSubcoreMesh(
        core_axis_name="c", subcore_axis_name="s",
        num_cores=_NC, num_subcores=_NS)
    cp = pltpu.CompilerParams(needs_layout_passes=False)
    parts = pl.kernel(
        functools.partial(_partials_body, num_classes, rpw),
        out_type=jax.ShapeDtypeStruct((_NW, 2, _L), jnp.float32),
        mesh=mesh,
        compiler_params=cp,
        scratch_types=[
            pltpu.VMEM((rpw,), jnp.int32),
            pltpu.VMEM((rpw // _CH, _CH), jnp.int32),
            pltpu.VMEM((rpw * 4,), jnp.float32),
            pltpu.VMEM((_NBUF, _CH, 128), jnp.float32),
            pltpu.VMEM((2, _L), jnp.float32),
            pltpu.SemaphoreType.DMA,
            pltpu.SemaphoreType.DMA,
        ],
    )(yp, yt, tci)
    res = pl.kernel(
        _reduce_body,
        out_type=jax.ShapeDtypeStruct((_L,), jnp.float32),
        mesh=mesh,
        compiler_params=cp,
        scratch_types=[
            pltpu.VMEM((_NW, 2, _L), jnp.float32),
            pltpu.VMEM((_L,), jnp.float32),
        ],
    )(parts)
    return res[0]


def kernel(y_true, y_pred, target_class_ids):
    B, R, C, _ = y_pred.shape
    N = B * R
    yp = y_pred.reshape(N * C * 4 // 128, 128)
    yt = y_true.reshape(N * 4)
    tci = target_class_ids.reshape(N)
    return _sc_loss(yp, yt, tci, N, C)


# speedup vs baseline: 65.8153x; 65.8153x over previous
import functools

import jax
import jax.numpy as jnp
from jax import lax
from jax.experimental import pallas as pl
from jax.experimental.pallas import tpu as pltpu
from jax.experimental.pallas import tpu_sc as plsc

_NC = 2
_NS = 16
_L = 16
_NW = _NC * _NS
_G = 128


def _partials_body(num_classes, rpw, yp_hbm, yt_hbm, tci_hbm, out_hbm,
                   tci_v, idx_v, yt_v, bufs, acc_v, sem0, sem1):
    c = lax.axis_index("c")
    s = lax.axis_index("s")
    wid = s * _NC + c
    ngrp = rpw // _G
    nch = ngrp * 4

    pltpu.sync_copy(tci_hbm.at[wid], tci_v)
    pltpu.sync_copy(yt_hbm.at[wid], yt_v)

    lanes = lax.iota(jnp.int32, _L)
    sems = [sem0, sem1]

    def build(i, carry):
        g = i // 8
        i8 = i % 8
        t16 = tci_v[pl.ds(i * _L, _L)]
        base16 = ((wid * num_classes + t16) * ngrp + g) * 4
        for j in range(4):
            idx_v[g * 4 + j, pl.ds(i8 * _L, _L)] = base16 + j
        return carry

    lax.fori_loop(0, rpw // _L, build, 0)

    def fire(k):
        pltpu.async_copy(yp_hbm.at[idx_v.at[k]], bufs.at[k % 2],
                         sems[k % 2])

    def drain(k):
        pltpu.make_async_copy(yp_hbm.at[idx_v.at[k]], bufs.at[k % 2],
                              sems[k % 2]).wait()

    fire(0)
    lacc = jnp.zeros((_L,), jnp.float32)
    cacc = jnp.zeros((_L,), jnp.float32)
    for k in range(nch):
        if k + 1 < nch:
            fire(k + 1)
        drain(k)
        buf = bufs.at[k % 2]
        g, j = k // 4, k % 4

        def step(i, carry, g=g, j=j, buf=buf):
            la, ca = carry
            r16 = i * _L + lanes
            pb16 = plsc.load_gather(buf, [r16, r16])
            tb16 = yt_v[j, pl.ds(g * _G + i * _L, _L)]
            t16 = tci_v[pl.ds(g * _G + i * _L, _L)]
            diff = jnp.abs(tb16 - pb16)
            loss = jnp.where(diff < 1.0, 0.5 * diff * diff, diff - 0.5)
            m = t16 > 0
            la = la + jnp.where(m, loss, 0.0)
            ca = ca + jnp.where(m, 1.0, 0.0)
            return la, ca

        lacc, cacc = lax.fori_loop(0, _G // _L, step, (lacc, cacc))

    acc_v[0, :] = lacc
    acc_v[1, :] = cacc
    pltpu.sync_copy(acc_v, out_hbm.at[wid])


def _reduce_body(parts_hbm, out_hbm, parts_v, res_v):
    c = lax.axis_index("c")
    s = lax.axis_index("s")

    @pl.when(jnp.logical_and(s == 0, c == 0))
    def _():
        pltpu.sync_copy(parts_hbm, parts_v)
        zero = jnp.zeros((_L,), jnp.float32)

        def red(i, carry):
            ls, cs = carry
            return ls + parts_v[i, 0, :], cs + parts_v[i, 1, :]

        ls, cs = lax.fori_loop(0, _NW, red, (zero, zero))
        tv = zero + jnp.sum(ls)
        cv = zero + jnp.sum(cs)
        mean = tv / jnp.maximum(cv, 1.0)
        res_v[:] = jnp.where(cv > 0.0, mean, zero)
        pltpu.sync_copy(res_v, out_hbm)


@functools.partial(jax.jit, static_argnums=(3, 4))
def _sc_loss(yp, yt, tci, num_rows, num_classes):
    rpw = num_rows // _NW
    mesh = plsc.VectorSubcoreMesh(
        core_axis_name="c", subcore_axis_name="s",
        num_cores=_NC, num_subcores=_NS)
    cp = pltpu.CompilerParams(needs_layout_passes=False)
    parts = pl.kernel(
        functools.partial(_partials_body, num_classes, rpw),
        out_type=jax.ShapeDtypeStruct((_NW, 2, _L), jnp.float32),
        mesh=mesh,
        compiler_params=cp,
        scratch_types=[
            pltpu.VMEM((rpw,), jnp.int32),
            pltpu.VMEM((rpw // _G * 4, _G), jnp.int32),
            pltpu.VMEM((4, rpw), jnp.float32),
            pltpu.VMEM((2, _G, 128), jnp.float32),
            pltpu.VMEM((2, _L), jnp.float32),
            pltpu.SemaphoreType.DMA,
            pltpu.SemaphoreType.DMA,
        ],
    )(yp, yt, tci)
    res = pl.kernel(
        _reduce_body,
        out_type=jax.ShapeDtypeStruct((_L,), jnp.float32),
        mesh=mesh,
        compiler_params=cp,
        scratch_types=[
            pltpu.VMEM((_NW, 2, _L), jnp.float32),
            pltpu.VMEM((_L,), jnp.float32),
        ],
    )(parts)
    return res[0]


def kernel(y_true, y_pred, target_class_ids):
    B, R, C, _ = y_pred.shape
    N = B * R
    yp = (y_pred.reshape(B, R // 128, 128, C, 4)
          .transpose(0, 3, 1, 4, 2)
          .reshape(B * C * (R // 128) * 4, 128))
    yt = y_true.transpose(0, 2, 1)
    return _sc_loss(yp, yt, target_class_ids, N, C)


# speedup vs baseline: 101.2749x; 1.5388x over previous
import functools

import jax
import jax.numpy as jnp
from jax import lax
from jax.experimental import pallas as pl
from jax.experimental.pallas import tpu as pltpu
from jax.experimental.pallas import tpu_sc as plsc

_NC = 2
_NS = 16
_L = 16
_NW = _NC * _NS
_G = 128


def _partials_body(num_classes, rpw, yp_hbm, yt_hbm, tci_hbm, out_hbm,
                   tci_v, idx_v, yt_v, bufs, acc_v, sem0, sem1):
    c = lax.axis_index("c")
    s = lax.axis_index("s")
    wid = s * _NC + c
    ngrp = rpw // _G
    nch = ngrp * 4

    pltpu.sync_copy(tci_hbm.at[wid], tci_v)
    pltpu.sync_copy(yt_hbm.at[wid], yt_v)

    lanes = lax.iota(jnp.int32, _L)
    sems = [sem0, sem1]

    def build(i, carry):
        g = i // 8
        i8 = i % 8
        t16 = tci_v[pl.ds(i * _L, _L)]
        base16 = (((wid * num_classes + t16) * ngrp + g) * 4) * 8 + i8
        for j in range(4):
            idx_v[g * 4 + j, pl.ds(i8 * _L, _L)] = base16 + j * 8
        return carry

    lax.fori_loop(0, rpw // _L, build, 0)

    def fire(k):
        pltpu.async_copy(yp_hbm.at[idx_v.at[k]], bufs.at[k % 2],
                         sems[k % 2])

    def drain(k):
        pltpu.make_async_copy(yp_hbm.at[idx_v.at[k]], bufs.at[k % 2],
                              sems[k % 2]).wait()

    fire(0)
    lacc = jnp.zeros((_L,), jnp.float32)
    cacc = jnp.zeros((_L,), jnp.float32)
    for k in range(nch):
        if k + 1 < nch:
            fire(k + 1)
        drain(k)
        buf = bufs.at[k % 2]
        g, j = k // 4, k % 4

        def step(i, carry, g=g, j=j, buf=buf):
            la, ca = carry
            r16 = i * _L + lanes
            pb16 = plsc.load_gather(buf, [r16, lanes])
            tb16 = yt_v[j, pl.ds(g * _G + i * _L, _L)]
            t16 = tci_v[pl.ds(g * _G + i * _L, _L)]
            diff = jnp.abs(tb16 - pb16)
            loss = jnp.where(diff < 1.0, 0.5 * diff * diff, diff - 0.5)
            m = t16 > 0
            la = la + jnp.where(m, loss, 0.0)
            ca = ca + jnp.where(m, 1.0, 0.0)
            return la, ca

        lacc, cacc = lax.fori_loop(0, _G // _L, step, (lacc, cacc))

    acc_v[0, :] = lacc
    acc_v[1, :] = cacc
    pltpu.sync_copy(acc_v, out_hbm.at[wid])


def _reduce_body(parts_hbm, out_hbm, parts_v, res_v):
    c = lax.axis_index("c")
    s = lax.axis_index("s")

    @pl.when(jnp.logical_and(s == 0, c == 0))
    def _():
        pltpu.sync_copy(parts_hbm, parts_v)
        zero = jnp.zeros((_L,), jnp.float32)

        def red(i, carry):
            ls, cs = carry
            return ls + parts_v[i, 0, :], cs + parts_v[i, 1, :]

        ls, cs = lax.fori_loop(0, _NW, red, (zero, zero))
        tv = zero + jnp.sum(ls)
        cv = zero + jnp.sum(cs)
        mean = tv / jnp.maximum(cv, 1.0)
        res_v[:] = jnp.where(cv > 0.0, mean, zero)
        pltpu.sync_copy(res_v, out_hbm)


@functools.partial(jax.jit, static_argnums=(3, 4))
def _sc_loss(yp, yt, tci, num_rows, num_classes):
    rpw = num_rows // _NW
    mesh = plsc.VectorSubcoreMesh(
        core_axis_name="c", subcore_axis_name="s",
        num_cores=_NC, num_subcores=_NS)
    cp = pltpu.CompilerParams(
        needs_layout_passes=False, use_tc_tiling_on_sc=False)
    parts = pl.kernel(
        functools.partial(_partials_body, num_classes, rpw),
        out_type=jax.ShapeDtypeStruct((_NW, 2, _L), jnp.float32),
        mesh=mesh,
        compiler_params=cp,
        scratch_types=[
            pltpu.VMEM((rpw,), jnp.int32),
            pltpu.VMEM((rpw // _G * 4, _G), jnp.int32),
            pltpu.VMEM((4, rpw), jnp.float32),
            pltpu.VMEM((2, _G, _L), jnp.float32),
            pltpu.VMEM((2, _L), jnp.float32),
            pltpu.SemaphoreType.DMA,
            pltpu.SemaphoreType.DMA,
        ],
    )(yp, yt, tci)
    res = pl.kernel(
        _reduce_body,
        out_type=jax.ShapeDtypeStruct((_L,), jnp.float32),
        mesh=mesh,
        compiler_params=cp,
        scratch_types=[
            pltpu.VMEM((_NW, 2, _L), jnp.float32),
            pltpu.VMEM((_L,), jnp.float32),
        ],
    )(parts)
    return res[0]


def kernel(y_true, y_pred, target_class_ids):
    B, R, C, _ = y_pred.shape
    N = B * R
    yp = (y_pred.reshape(B, R // 128, 128, C, 4)
          .transpose(0, 3, 1, 4, 2)
          .reshape(B * C * (R // 128) * 4 * 8, 16))
    yt = y_true.transpose(0, 2, 1)
    return _sc_loss(yp, yt, target_class_ids, N, C)
